# baseline (speedup 1.0000x reference)
import functools

import jax
import jax.numpy as jnp
from jax import lax
from jax.experimental import pallas as pl
from jax.experimental.pallas import tpu as pltpu
from jax.experimental.pallas import tpu_sc as plsc

_N_E = 8192
_E_DIM = 64
_BETA = 0.25
_M = 4608
_BM = 512
_W = 1024

_NW = 32
_BPW = _M // _NW
_CH = 72
_NCH = _BPW // _CH


def _vq_tc_body(z_ref, e_ref, idx_ref, loss_ref):
    z = z_ref[...]
    z_sq = jnp.sum(z * z, axis=1, keepdims=True)

    def chunk_d2(c):
        ec = e_ref[pl.ds(c * _W, _W), :]
        e_sq = jnp.sum(ec * ec, axis=1)[None, :]
        mm = lax.dot_general(z, ec, (((1,), (1,)), ((), ())),
                             preferred_element_type=jnp.float32)
        return z_sq + e_sq - 2.0 * mm

    lanes = lax.broadcasted_iota(jnp.int32, (_BM, _W), 1).astype(jnp.float32)
    runv = chunk_d2(0)
    runi = lanes
    for c in range(1, _N_E // _W):
        d2c = chunk_d2(c)
        lt = d2c < runv
        runi = jnp.where(lt, lanes + jnp.float32(c * _W), runi)
        runv = jnp.minimum(runv, d2c)

    minv = jnp.min(runv, axis=1, keepdims=True)
    cand = jnp.where(runv == minv, runi, jnp.float32(_N_E))
    idx = jnp.min(cand, axis=1).astype(jnp.int32)
    part = jnp.sum(jnp.maximum(minv, 0.0))
    idx_ref[...] = idx

    @pl.when(pl.program_id(0) == 0)
    def _():
        loss_ref[...] = jnp.zeros_like(loss_ref)

    loss_ref[...] += part.reshape(1, 1)


@functools.cache
def _sc_gather_kernel():
    mesh = plsc.VectorSubcoreMesh(core_axis_name="c", subcore_axis_name="s")

    @functools.partial(
        pl.kernel,
        out_type=jax.ShapeDtypeStruct((_M, _E_DIM), jnp.float32),
        mesh=mesh,
        scratch_types=[
            pltpu.VMEM((_BPW,), jnp.int32),
            pltpu.VMEM((_CH, _E_DIM), jnp.float32),
            pltpu.SemaphoreType.DMA,
        ],
        compiler_params=pltpu.CompilerParams(use_tc_tiling_on_sc=False),
    )
    def _sc_gather(table_hbm, idx_hbm, out_hbm, idx_v, rows_v, sem):
        wid = lax.axis_index("s") * 2 + lax.axis_index("c")
        base = wid * _BPW
        pltpu.sync_copy(idx_hbm.at[pl.ds(base, _BPW)], idx_v)
        for j in range(_NCH):
            pltpu.async_copy(table_hbm.at[idx_v.at[pl.ds(j * _CH, _CH)]],
                             rows_v, sem).wait()
            pltpu.sync_copy(rows_v, out_hbm.at[pl.ds(base + j * _CH, _CH)])

    return _sc_gather


def kernel(z, embedding_weight):
    zt = jnp.transpose(z, (0, 2, 3, 1))
    z_flat = zt.reshape(-1, _E_DIM)
    idx_flat, loss_sum = pl.pallas_call(
        _vq_tc_body,
        grid=(_M // _BM,),
        in_specs=[
            pl.BlockSpec((_BM, _E_DIM), lambda i: (i, 0)),
            pl.BlockSpec((_N_E, _E_DIM), lambda i: (0, 0)),
        ],
        out_specs=[
            pl.BlockSpec((_BM,), lambda i: (i,)),
            pl.BlockSpec((1, 1), lambda i: (0, 0)),
        ],
        out_shape=[
            jax.ShapeDtypeStruct((_M,), jnp.int32),
            jax.ShapeDtypeStruct((1, 1), jnp.float32),
        ],
    )(z_flat, embedding_weight)

    z_q_flat = _sc_gather_kernel()(embedding_weight, idx_flat)
    z_q = z_q_flat.reshape(zt.shape)

    m = loss_sum[0, 0] / float(_M * _E_DIM)
    loss = _BETA * m + m
    out = jnp.transpose(z_q, (0, 3, 1, 2))
    idx_out = idx_flat.reshape(zt.shape[:-1])
    return out, loss, idx_out
